# baseline (speedup 1.0000x reference)
import functools

import jax
import jax.numpy as jnp
from jax import lax
from jax.experimental import pallas as pl
from jax.experimental.pallas import tpu as pltpu
from jax.experimental.pallas import tpu_sc as plsc

H = 260
W = 346
HW = H * W
VOL = 2 * HW
N = 1048576
NC = 2
NS = 16
NW = NC * NS
NLOC = N // NW
C = 2048
NCHUNK = NLOC // C
NPAIR = NCHUNK // 2
SLICE = 11264
RHALF = SLICE // 2
VOLP = SLICE * NS
FCOL = H * (H - 1) + (W - 1) + 1
FSLICE = 4232
FPAD = FSLICE * NS


def _sc_body(wh_hbm, tp_hbm, flow_hbm, scal_hbm, zeros_hbm,
             wsum_hbm, tsum_hbm,
             whvA, tpvA, whvB, tpvB,
             fivA, fivB, gbufA, gbufB,
             idxA, vwA, vwtA, idxB, vwB, vwtB,
             scalv,
             semEvA, semEvB, semGA, semGB, semScA, semScB,
             flow_sh, accw_sh, acct_sh):
    cid = lax.axis_index("c")
    sid = lax.axis_index("s")
    wid = sid * NC + cid
    sl = pl.ds(sid * SLICE, SLICE)

    pltpu.sync_copy(scal_hbm, scalv)

    fsl = pl.ds(sid * FSLICE, FSLICE)
    fstg = idxA.at[pl.ds(0, FSLICE)]
    pltpu.sync_copy(flow_hbm.at[fsl], fstg)
    pltpu.sync_copy(fstg, flow_sh.at[fsl])

    zb = vwtA.at[pl.ds(0, RHALF)]
    pltpu.sync_copy(zeros_hbm, zb)
    for k in range(2):
        asl = pl.ds(sid * SLICE + k * RHALF, RHALF)
        pltpu.sync_copy(zb, accw_sh.at[asl])
        pltpu.sync_copy(zb, acct_sh.at[asl])

    plsc.subcore_barrier()

    t0 = scalv[0, :]
    invtl = scalv[1, :]
    evbase = wid * NLOC

    def start_ev(ci, bufs, sem):
        whv, tpv = bufs
        s = pl.ds(evbase + ci * C, C)
        pltpu.async_copy(wh_hbm.at[s], whv, sem)
        pltpu.async_copy(tp_hbm.at[s], tpv, sem)

    def wait_ev(ci, bufs, sem):
        whv, tpv = bufs
        s = pl.ds(evbase + ci * C, C)
        pltpu.make_async_copy(wh_hbm.at[s], whv, sem).wait()
        pltpu.make_async_copy(tp_hbm.at[s], tpv, sem).wait()

    def fi_pass(bufs, fiv):
        whv, tpv = bufs

        def _fi(j, _):
            s = pl.ds(j * 16, 16)
            wh = whv[s]
            fiv[s] = H * (wh >> 16) + (wh & jnp.int32(0xFFFF))
            return 0
        lax.fori_loop(0, C // 16, _fi, 0)

    def issue_gather(fiv, gbuf, sem):
        pltpu.async_copy(flow_sh.at[fiv], gbuf, sem)

    def wait_gather(fiv, gbuf, sem):
        pltpu.make_async_copy(flow_sh.at[fiv], gbuf, sem).wait()

    def issue_scatter(sbufs, sem):
        idxb, vwb, vwtb = sbufs
        pltpu.async_copy(vwb, accw_sh.at[idxb], sem, add=True)
        pltpu.async_copy(vwtb, acct_sh.at[idxb], sem, add=True)

    def wait_scatter(sbufs, sem):
        idxb, vwb, vwtb = sbufs
        pltpu.make_async_copy(vwb, accw_sh.at[idxb], sem).wait()
        pltpu.make_async_copy(vwtb, acct_sh.at[idxb], sem).wait()

    def main_pass(bufs, gbuf, sbufs):
        whv, tpv = bufs
        idxb, vwb, vwtb = sbufs

        def _main(j, _):
            s = pl.ds(j * 16, 16)
            wh = whv[s]
            tp = tpv[s]
            wf = (wh & jnp.int32(0xFFFF)).astype(jnp.float32)
            hf = (wh >> 16).astype(jnp.float32)
            tf = lax.bitcast_convert_type(tp & jnp.int32(0x7FFFFFFF),
                                          jnp.float32)
            tr = tf - t0
            g = gbuf[s]
            f0 = lax.bitcast_convert_type(g << 16, jnp.float32)
            f1 = lax.bitcast_convert_type(g & jnp.int32(-65536), jnp.float32)
            delt = 1.0 - tr * invtl
            wr = wf + delt * f0
            hr = hf + delt * f1

            wi0 = wr.astype(jnp.int32)
            wf0 = wi0.astype(jnp.float32)
            wfl_i = jnp.where(wf0 > wr, wi0 - 1, wi0)
            wfl_f = jnp.where(wf0 > wr, wf0 - 1.0, wf0)
            dw_ce = wr - wfl_f
            dw_fl = (wfl_f + 1.0) - wr
            wce_i = jnp.where(wr != wfl_f, wfl_i + 1, wfl_i)
            wfl = jnp.clip(wfl_i, 0, W - 1)
            wce = jnp.clip(wce_i, 0, W - 1)

            hi0 = hr.astype(jnp.int32)
            hf0 = hi0.astype(jnp.float32)
            hfl_i = jnp.where(hf0 > hr, hi0 - 1, hi0)
            hfl_f = jnp.where(hf0 > hr, hf0 - 1.0, hf0)
            dh_ce = hr - hfl_f
            dh_fl = (hfl_f + 1.0) - hr
            hce_i = jnp.where(hr != hfl_f, hfl_i + 1, hfl_i)
            hfl = jnp.clip(hfl_i, 0, H - 1)
            hce = jnp.clip(hce_i, 0, H - 1)

            vbase = jnp.where(tp >= 0, HW - W, -W).astype(jnp.int32)
            corners = ((wfl, dw_fl, hfl, dh_fl),
                       (wfl, dw_fl, hce, dh_ce),
                       (wce, dw_ce, hfl, dh_fl),
                       (wce, dw_ce, hce, dh_ce))
            for k, (wc, dwc, hc, dhc) in enumerate(corners):
                ind = vbase + W * hc + wc
                ind = jnp.where(ind < 0, ind + VOL, ind)
                vw = dwc * dhc
                ks = pl.ds(k * C + j * 16, 16)
                idxb[ks] = ind
                vwb[ks] = vw
                vwtb[ks] = vw * tr
            return 0
        lax.fori_loop(0, C // 16, _main, 0)

    bufsA = (whvA, tpvA)
    bufsB = (whvB, tpvB)
    sbufsA = (idxA, vwA, vwtA)
    sbufsB = (idxB, vwB, vwtB)

    start_ev(0, bufsA, semEvA)
    wait_ev(0, bufsA, semEvA)
    fi_pass(bufsA, fivA)
    issue_gather(fivA, gbufA, semGA)
    start_ev(1, bufsB, semEvB)

    def _pair(ci2, _):
        c0i = 2 * ci2
        c1i = 2 * ci2 + 1
        wait_gather(fivA, gbufA, semGA)
        wait_ev(c1i, bufsB, semEvB)
        fi_pass(bufsB, fivB)
        issue_gather(fivB, gbufB, semGB)

        @pl.when(ci2 > 0)
        def _():
            wait_scatter(sbufsA, semScA)
        main_pass(bufsA, gbufA, sbufsA)
        issue_scatter(sbufsA, semScA)

        @pl.when(ci2 < NPAIR - 1)
        def _():
            start_ev(c0i + 2, bufsA, semEvA)

        wait_gather(fivB, gbufB, semGB)

        @pl.when(ci2 < NPAIR - 1)
        def _():
            wait_ev(c0i + 2, bufsA, semEvA)
            fi_pass(bufsA, fivA)
            issue_gather(fivA, gbufA, semGA)

        @pl.when(ci2 > 0)
        def _():
            wait_scatter(sbufsB, semScB)
        main_pass(bufsB, gbufB, sbufsB)
        issue_scatter(sbufsB, semScB)

        @pl.when(ci2 < NPAIR - 1)
        def _():
            start_ev(c1i + 2, bufsB, semEvB)
        return 0
    lax.fori_loop(0, NPAIR, _pair, 0)

    wait_scatter(sbufsA, semScA)
    wait_scatter(sbufsB, semScB)

    plsc.subcore_barrier()

    for k in range(2):
        asl = pl.ds(sid * SLICE + k * RHALF, RHALF)
        stw = vwA.at[pl.ds(0, RHALF)]
        stt = vwtA.at[pl.ds(0, RHALF)]
        pltpu.sync_copy(accw_sh.at[asl], stw)
        pltpu.sync_copy(stw, wsum_hbm.at[cid, asl])
        pltpu.sync_copy(acct_sh.at[asl], stt)
        pltpu.sync_copy(stt, tsum_hbm.at[cid, asl])


_sc_kernel = functools.partial(
    pl.kernel,
    out_type=(jax.ShapeDtypeStruct((NC, VOLP), jnp.float32),
              jax.ShapeDtypeStruct((NC, VOLP), jnp.float32)),
    mesh=plsc.VectorSubcoreMesh(core_axis_name="c", subcore_axis_name="s"),
    scratch_types=[
        pltpu.VMEM((C,), jnp.int32),
        pltpu.VMEM((C,), jnp.int32),
        pltpu.VMEM((C,), jnp.int32),
        pltpu.VMEM((C,), jnp.int32),
        pltpu.VMEM((C,), jnp.int32),
        pltpu.VMEM((C,), jnp.int32),
        pltpu.VMEM((C,), jnp.int32),
        pltpu.VMEM((C,), jnp.int32),
        pltpu.VMEM((4 * C,), jnp.int32),
        pltpu.VMEM((4 * C,), jnp.float32),
        pltpu.VMEM((4 * C,), jnp.float32),
        pltpu.VMEM((4 * C,), jnp.int32),
        pltpu.VMEM((4 * C,), jnp.float32),
        pltpu.VMEM((4 * C,), jnp.float32),
        pltpu.VMEM((2, 16), jnp.float32),
        pltpu.SemaphoreType.DMA,
        pltpu.SemaphoreType.DMA,
        pltpu.SemaphoreType.DMA,
        pltpu.SemaphoreType.DMA,
        pltpu.SemaphoreType.DMA,
        pltpu.SemaphoreType.DMA,
        pltpu.VMEM_SHARED((FPAD,), jnp.int32),
        pltpu.VMEM_SHARED((VOLP,), jnp.float32),
        pltpu.VMEM_SHARED((VOLP,), jnp.float32),
    ],
)(_sc_body)


def _combine_body(w_ref, t_ref, o_ref):
    w2 = w_ref[...]
    t2 = t_ref[...]
    o_ref[0, 0] = jnp.sum((t2[0] + t2[1]) / (w2[0] + w2[1] + 1e-8))


_combine = pl.pallas_call(
    _combine_body,
    out_shape=jax.ShapeDtypeStruct((1, 1), jnp.float32),
    out_specs=pl.BlockSpec(memory_space=pltpu.SMEM),
)


def kernel(events, end_flow):
    wi = events[:, 0].astype(jnp.int32)
    hi = events[:, 1].astype(jnp.int32)
    wh = wi | (hi << 16)
    tu = jax.lax.bitcast_convert_type(events[:, 2], jnp.int32)
    tp = jnp.where(events[:, 3] > 0.0, tu, tu | jnp.int32(-2147483648))
    t0 = events[0, 2]
    tl = events[N - 1, 2] - t0
    scal = jnp.stack([jnp.full((16,), t0, jnp.float32),
                      jnp.full((16,), 1.0 / tl, jnp.float32)])
    fp = end_flow.reshape(2, HW)
    fb = fp[:, :FCOL].astype(jnp.bfloat16)
    fu = jax.lax.bitcast_convert_type(fb, jnp.uint16).astype(jnp.uint32)
    packed = jax.lax.bitcast_convert_type(fu[0] | (fu[1] << 16), jnp.int32)
    flow_pad = jnp.concatenate([packed,
                                jnp.zeros((FPAD - FCOL,), jnp.int32)])
    zeros = jnp.zeros((RHALF,), jnp.float32)
    wsum, tsum = _sc_kernel(wh, tp, flow_pad, scal, zeros)
    out = _combine(wsum.reshape(NC, VOLP // 128, 128),
                   tsum.reshape(NC, VOLP // 128, 128))
    return out[0, 0]


# speedup vs baseline: 1.1691x; 1.1691x over previous
import functools

import jax
import jax.numpy as jnp
from jax import lax
from jax.experimental import pallas as pl
from jax.experimental.pallas import tpu as pltpu
from jax.experimental.pallas import tpu_sc as plsc

H = 260
W = 346
HW = H * W
VOL = 2 * HW
N = 1048576
NC = 2
NS = 16
NW = NC * NS
NLOC = N // NW
C = 2048
NCHUNK = NLOC // C
NPAIR = NCHUNK // 2
SLICE = 11264
RHALF = SLICE // 2
VOLP = SLICE * NS
FCOL = H * (H - 1) + (W - 1) + 1
FSLICE = 4232
FPAD = FSLICE * NS


def _sc_body(w_hbm, h_hbm, t_hbm, p_hbm, flow_hbm, scal_hbm, zeros_hbm,
             wsum_hbm, tsum_hbm,
             wvA, hvA, tvA, pvA, wvB, hvB, tvB, pvB,
             fivA, fivB, gbufA, gbufB,
             idxA, vwA, vwtA, idxB, vwB, vwtB,
             scalv,
             semEvA, semEvB, semGA, semGB, semScA, semScB,
             flow_sh, accw_sh, acct_sh):
    cid = lax.axis_index("c")
    sid = lax.axis_index("s")
    wid = sid * NC + cid
    sl = pl.ds(sid * SLICE, SLICE)

    pltpu.sync_copy(scal_hbm, scalv)

    fsl = pl.ds(sid * FSLICE, FSLICE)
    fstg = idxA.at[pl.ds(0, FSLICE)]
    pltpu.sync_copy(flow_hbm.at[fsl], fstg)
    pltpu.sync_copy(fstg, flow_sh.at[fsl])

    zb = vwtA.at[pl.ds(0, RHALF)]
    pltpu.sync_copy(zeros_hbm, zb)
    for k in range(2):
        asl = pl.ds(sid * SLICE + k * RHALF, RHALF)
        pltpu.sync_copy(zb, accw_sh.at[asl])
        pltpu.sync_copy(zb, acct_sh.at[asl])

    plsc.subcore_barrier()

    t0 = scalv[0, :]
    invtl = scalv[1, :]
    evbase = wid * NLOC

    def start_ev(ci, bufs, sem):
        wv, hv, tv, pv = bufs
        s = pl.ds(evbase + ci * C, C)
        pltpu.async_copy(w_hbm.at[s], wv, sem)
        pltpu.async_copy(h_hbm.at[s], hv, sem)
        pltpu.async_copy(t_hbm.at[s], tv, sem)
        pltpu.async_copy(p_hbm.at[s], pv, sem)

    def wait_ev(ci, bufs, sem):
        wv, hv, tv, pv = bufs
        s = pl.ds(evbase + ci * C, C)
        pltpu.make_async_copy(w_hbm.at[s], wv, sem).wait()
        pltpu.make_async_copy(h_hbm.at[s], hv, sem).wait()
        pltpu.make_async_copy(t_hbm.at[s], tv, sem).wait()
        pltpu.make_async_copy(p_hbm.at[s], pv, sem).wait()

    def fi_pass(bufs, fiv):
        wv, hv, tv, pv = bufs

        def _fi(j, _):
            s = pl.ds(j * 16, 16)
            fiv[s] = H * hv[s].astype(jnp.int32) + wv[s].astype(jnp.int32)
            return 0
        lax.fori_loop(0, C // 16, _fi, 0)

    def issue_gather(fiv, gbuf, sem):
        pltpu.async_copy(flow_sh.at[fiv], gbuf, sem)

    def wait_gather(fiv, gbuf, sem):
        pltpu.make_async_copy(flow_sh.at[fiv], gbuf, sem).wait()

    def issue_scatter(sbufs, sem):
        idxb, vwb, vwtb = sbufs
        pltpu.async_copy(vwb, accw_sh.at[idxb], sem, add=True)
        pltpu.async_copy(vwtb, acct_sh.at[idxb], sem, add=True)

    def wait_scatter(sbufs, sem):
        idxb, vwb, vwtb = sbufs
        pltpu.make_async_copy(vwb, accw_sh.at[idxb], sem).wait()
        pltpu.make_async_copy(vwtb, acct_sh.at[idxb], sem).wait()

    def main_pass(bufs, gbuf, sbufs):
        wv, hv, tv, pv = bufs
        idxb, vwb, vwtb = sbufs

        def _main(j, _):
            s = pl.ds(j * 16, 16)
            wf = wv[s]
            hf = hv[s]
            tr = tv[s] - t0
            g = gbuf[s]
            f0 = lax.bitcast_convert_type(g << 16, jnp.float32)
            f1 = lax.bitcast_convert_type(g & jnp.int32(-65536), jnp.float32)
            delt = 1.0 - tr * invtl
            wr = wf + delt * f0
            hr = hf + delt * f1

            wi0 = wr.astype(jnp.int32)
            wf0 = wi0.astype(jnp.float32)
            wfl_i = jnp.where(wf0 > wr, wi0 - 1, wi0)
            wfl_f = jnp.where(wf0 > wr, wf0 - 1.0, wf0)
            dw_ce = wr - wfl_f
            dw_fl = (wfl_f + 1.0) - wr
            wce_i = jnp.where(wr != wfl_f, wfl_i + 1, wfl_i)
            wfl = jnp.clip(wfl_i, 0, W - 1)
            wce = jnp.clip(wce_i, 0, W - 1)

            hi0 = hr.astype(jnp.int32)
            hf0 = hi0.astype(jnp.float32)
            hfl_i = jnp.where(hf0 > hr, hi0 - 1, hi0)
            hfl_f = jnp.where(hf0 > hr, hf0 - 1.0, hf0)
            dh_ce = hr - hfl_f
            dh_fl = (hfl_f + 1.0) - hr
            hce_i = jnp.where(hr != hfl_f, hfl_i + 1, hfl_i)
            hfl = jnp.clip(hfl_i, 0, H - 1)
            hce = jnp.clip(hce_i, 0, H - 1)

            vbase = jnp.where(pv[s] > 0.0, HW - W, -W).astype(jnp.int32)
            corners = ((wfl, dw_fl, hfl, dh_fl),
                       (wfl, dw_fl, hce, dh_ce),
                       (wce, dw_ce, hfl, dh_fl),
                       (wce, dw_ce, hce, dh_ce))
            for k, (wc, dwc, hc, dhc) in enumerate(corners):
                ind = vbase + W * hc + wc
                ind = jnp.where(ind < 0, ind + VOL, ind)
                vw = dwc * dhc
                ks = pl.ds(k * C + j * 16, 16)
                idxb[ks] = ind
                vwb[ks] = vw
                vwtb[ks] = vw * tr
            return 0
        lax.fori_loop(0, C // 16, _main, 0)

    bufsA = (wvA, hvA, tvA, pvA)
    bufsB = (wvB, hvB, tvB, pvB)
    sbufsA = (idxA, vwA, vwtA)
    sbufsB = (idxB, vwB, vwtB)

    start_ev(0, bufsA, semEvA)
    wait_ev(0, bufsA, semEvA)
    fi_pass(bufsA, fivA)
    issue_gather(fivA, gbufA, semGA)
    start_ev(1, bufsB, semEvB)

    def _pair(ci2, _):
        c0i = 2 * ci2
        c1i = 2 * ci2 + 1
        wait_gather(fivA, gbufA, semGA)
        wait_ev(c1i, bufsB, semEvB)
        fi_pass(bufsB, fivB)
        issue_gather(fivB, gbufB, semGB)

        @pl.when(ci2 > 0)
        def _():
            wait_scatter(sbufsA, semScA)
        main_pass(bufsA, gbufA, sbufsA)
        issue_scatter(sbufsA, semScA)

        @pl.when(ci2 < NPAIR - 1)
        def _():
            start_ev(c0i + 2, bufsA, semEvA)

        wait_gather(fivB, gbufB, semGB)

        @pl.when(ci2 < NPAIR - 1)
        def _():
            wait_ev(c0i + 2, bufsA, semEvA)
            fi_pass(bufsA, fivA)
            issue_gather(fivA, gbufA, semGA)

        @pl.when(ci2 > 0)
        def _():
            wait_scatter(sbufsB, semScB)
        main_pass(bufsB, gbufB, sbufsB)
        issue_scatter(sbufsB, semScB)

        @pl.when(ci2 < NPAIR - 1)
        def _():
            start_ev(c1i + 2, bufsB, semEvB)
        return 0
    lax.fori_loop(0, NPAIR, _pair, 0)

    wait_scatter(sbufsA, semScA)
    wait_scatter(sbufsB, semScB)

    plsc.subcore_barrier()

    for k in range(2):
        asl = pl.ds(sid * SLICE + k * RHALF, RHALF)
        stw = vwA.at[pl.ds(0, RHALF)]
        stt = vwtA.at[pl.ds(0, RHALF)]
        pltpu.sync_copy(accw_sh.at[asl], stw)
        pltpu.sync_copy(stw, wsum_hbm.at[cid, asl])
        pltpu.sync_copy(acct_sh.at[asl], stt)
        pltpu.sync_copy(stt, tsum_hbm.at[cid, asl])


_sc_kernel = functools.partial(
    pl.kernel,
    out_type=(jax.ShapeDtypeStruct((NC, VOLP), jnp.float32),
              jax.ShapeDtypeStruct((NC, VOLP), jnp.float32)),
    mesh=plsc.VectorSubcoreMesh(core_axis_name="c", subcore_axis_name="s"),
    scratch_types=[
        pltpu.VMEM((C,), jnp.float32),
        pltpu.VMEM((C,), jnp.float32),
        pltpu.VMEM((C,), jnp.float32),
        pltpu.VMEM((C,), jnp.float32),
        pltpu.VMEM((C,), jnp.float32),
        pltpu.VMEM((C,), jnp.float32),
        pltpu.VMEM((C,), jnp.float32),
        pltpu.VMEM((C,), jnp.float32),
        pltpu.VMEM((C,), jnp.int32),
        pltpu.VMEM((C,), jnp.int32),
        pltpu.VMEM((C,), jnp.int32),
        pltpu.VMEM((C,), jnp.int32),
        pltpu.VMEM((4 * C,), jnp.int32),
        pltpu.VMEM((4 * C,), jnp.float32),
        pltpu.VMEM((4 * C,), jnp.float32),
        pltpu.VMEM((4 * C,), jnp.int32),
        pltpu.VMEM((4 * C,), jnp.float32),
        pltpu.VMEM((4 * C,), jnp.float32),
        pltpu.VMEM((2, 16), jnp.float32),
        pltpu.SemaphoreType.DMA,
        pltpu.SemaphoreType.DMA,
        pltpu.SemaphoreType.DMA,
        pltpu.SemaphoreType.DMA,
        pltpu.SemaphoreType.DMA,
        pltpu.SemaphoreType.DMA,
        pltpu.VMEM_SHARED((FPAD,), jnp.int32),
        pltpu.VMEM_SHARED((VOLP,), jnp.float32),
        pltpu.VMEM_SHARED((VOLP,), jnp.float32),
    ],
)(_sc_body)


def _combine_body(w_ref, t_ref, o_ref):
    w2 = w_ref[...]
    t2 = t_ref[...]
    o_ref[0, 0] = jnp.sum((t2[0] + t2[1]) / (w2[0] + w2[1] + 1e-8))


_combine = pl.pallas_call(
    _combine_body,
    out_shape=jax.ShapeDtypeStruct((1, 1), jnp.float32),
    out_specs=pl.BlockSpec(memory_space=pltpu.SMEM),
)


def kernel(events, end_flow):
    w = events[:, 0]
    h = events[:, 1]
    t = events[:, 2]
    p = events[:, 3]
    t0 = events[0, 2]
    tl = events[N - 1, 2] - t0
    scal = jnp.stack([jnp.full((16,), t0, jnp.float32),
                      jnp.full((16,), 1.0 / tl, jnp.float32)])
    fp = end_flow.reshape(2, HW)
    fb = fp[:, :FCOL].astype(jnp.bfloat16)
    fu = jax.lax.bitcast_convert_type(fb, jnp.uint16).astype(jnp.uint32)
    packed = jax.lax.bitcast_convert_type(fu[0] | (fu[1] << 16), jnp.int32)
    flow_pad = jnp.concatenate([packed,
                                jnp.zeros((FPAD - FCOL,), jnp.int32)])
    zeros = jnp.zeros((RHALF,), jnp.float32)
    wsum, tsum = _sc_kernel(w, h, t, p, flow_pad, scal, zeros)
    out = _combine(wsum.reshape(NC, VOLP // 128, 128),
                   tsum.reshape(NC, VOLP // 128, 128))
    return out[0, 0]


# speedup vs baseline: 1.1727x; 1.0031x over previous
import functools

import jax
import jax.numpy as jnp
from jax import lax
from jax.experimental import pallas as pl
from jax.experimental.pallas import tpu as pltpu
from jax.experimental.pallas import tpu_sc as plsc

H = 260
W = 346
HW = H * W
VOL = 2 * HW
N = 1048576
NC = 2
NS = 16
NW = NC * NS
NLOC = N // NW
C = 2048
NCHUNK = NLOC // C
NPAIR = NCHUNK // 2
SLICE = 11264
RHALF = SLICE // 2
VOLP = SLICE * NS
FCOL = H * (H - 1) + (W - 1) + 1
FSLICE = 4232
FPAD = FSLICE * NS


def _sc_body(w_hbm, h_hbm, t_hbm, p_hbm, flow_hbm, scal_hbm, zeros_hbm,
             wsum_hbm, tsum_hbm,
             wvA, hvA, tvA, pvA, wvB, hvB, tvB, pvB,
             fivA, fivB, gbufA, gbufB,
             idxA, vwA, vwtA, idxB, vwB, vwtB,
             scalv,
             semEvA, semEvB, semGA, semGB, semScA, semScB,
             flow_sh, accw_sh, acct_sh):
    cid = lax.axis_index("c")
    sid = lax.axis_index("s")
    wid = sid * NC + cid

    pltpu.sync_copy(scal_hbm, scalv)

    fsl = pl.ds(sid * FSLICE, FSLICE)
    fstg = idxA.at[pl.ds(0, FSLICE)]
    pltpu.sync_copy(flow_hbm.at[fsl], fstg)
    pltpu.sync_copy(fstg, flow_sh.at[fsl])

    zb = vwtA.at[pl.ds(0, RHALF)]
    pltpu.sync_copy(zeros_hbm, zb)
    for k in range(2):
        asl = pl.ds(sid * SLICE + k * RHALF, RHALF)
        pltpu.sync_copy(zb, accw_sh.at[asl])
        pltpu.sync_copy(zb, acct_sh.at[asl])

    plsc.subcore_barrier()

    t0 = scalv[0, :]
    invtl = scalv[1, :]
    evbase = wid * NLOC

    def start_ev(ci, bufs, sem):
        wv, hv, tv, pv = bufs
        s = pl.ds(evbase + ci * C, C)
        pltpu.async_copy(w_hbm.at[s], wv, sem)
        pltpu.async_copy(h_hbm.at[s], hv, sem)
        pltpu.async_copy(t_hbm.at[s], tv, sem)
        pltpu.async_copy(p_hbm.at[s], pv, sem)

    def wait_ev(ci, bufs, sem):
        wv, hv, tv, pv = bufs
        s = pl.ds(evbase + ci * C, C)
        pltpu.make_async_copy(w_hbm.at[s], wv, sem).wait()
        pltpu.make_async_copy(h_hbm.at[s], hv, sem).wait()
        pltpu.make_async_copy(t_hbm.at[s], tv, sem).wait()
        pltpu.make_async_copy(p_hbm.at[s], pv, sem).wait()

    def fi_pass(bufs, fiv):
        wv, hv, tv, pv = bufs

        def _fi(j, _):
            s = pl.ds(j * 16, 16)
            fiv[s] = H * hv[s].astype(jnp.int32) + wv[s].astype(jnp.int32)
            return 0
        lax.fori_loop(0, C // 16, _fi, 0)

    def issue_gather(fiv, gbuf, sem):
        pltpu.async_copy(flow_sh.at[fiv], gbuf, sem)

    def wait_gather(fiv, gbuf, sem):
        pltpu.make_async_copy(flow_sh.at[fiv], gbuf, sem).wait()

    def issue_scatter(sbufs, sem):
        idxb, vwb, vwtb = sbufs
        pltpu.async_copy(vwb, accw_sh.at[idxb], sem, add=True)
        pltpu.async_copy(vwtb, acct_sh.at[idxb], sem, add=True)

    def wait_scatter(sbufs, sem):
        idxb, vwb, vwtb = sbufs
        pltpu.make_async_copy(vwb, accw_sh.at[idxb], sem).wait()
        pltpu.make_async_copy(vwtb, acct_sh.at[idxb], sem).wait()

    def main_pass(bufs, gbuf, sbufs):
        wv, hv, tv, pv = bufs
        idxb, vwb, vwtb = sbufs

        def _main(j, _):
            s = pl.ds(j * 16, 16)
            wf = wv[s]
            hf = hv[s]
            tr = tv[s] - t0
            g = gbuf[s]
            f0 = lax.bitcast_convert_type(g << 16, jnp.float32)
            f1 = lax.bitcast_convert_type(g & jnp.int32(-65536), jnp.float32)
            delt = 1.0 - tr * invtl
            wr = wf + delt * f0
            hr = hf + delt * f1

            wi0 = wr.astype(jnp.int32)
            wf0 = wi0.astype(jnp.float32)
            wfl_i = jnp.where(wf0 > wr, wi0 - 1, wi0)
            wfl_f = jnp.where(wf0 > wr, wf0 - 1.0, wf0)
            dw_ce = wr - wfl_f
            dw_fl = (wfl_f + 1.0) - wr
            wce_i = jnp.where(wr != wfl_f, wfl_i + 1, wfl_i)
            wfl = jnp.clip(wfl_i, 0, W - 1)
            wce = jnp.clip(wce_i, 0, W - 1)

            hi0 = hr.astype(jnp.int32)
            hf0 = hi0.astype(jnp.float32)
            hfl_i = jnp.where(hf0 > hr, hi0 - 1, hi0)
            hfl_f = jnp.where(hf0 > hr, hf0 - 1.0, hf0)
            dh_ce = hr - hfl_f
            dh_fl = (hfl_f + 1.0) - hr
            hce_i = jnp.where(hr != hfl_f, hfl_i + 1, hfl_i)
            hfl = jnp.clip(hfl_i, 0, H - 1)
            hce = jnp.clip(hce_i, 0, H - 1)

            vbase = jnp.where(pv[s] > 0.0, HW - W, -W).astype(jnp.int32)
            corners = ((wfl, dw_fl, hfl, dh_fl),
                       (wfl, dw_fl, hce, dh_ce),
                       (wce, dw_ce, hfl, dh_fl),
                       (wce, dw_ce, hce, dh_ce))
            for k, (wc, dwc, hc, dhc) in enumerate(corners):
                ind = vbase + W * hc + wc
                ind = jnp.where(ind < 0, ind + VOL, ind)
                vw = dwc * dhc
                ks = pl.ds(k * C + j * 16, 16)
                idxb[ks] = ind
                vwb[ks] = vw
                vwtb[ks] = vw * tr
            return 0
        lax.fori_loop(0, C // 16, _main, 0)

    bufsA = (wvA, hvA, tvA, pvA)
    bufsB = (wvB, hvB, tvB, pvB)
    sbufsA = (idxA, vwA, vwtA)
    sbufsB = (idxB, vwB, vwtB)

    start_ev(0, bufsA, semEvA)
    wait_ev(0, bufsA, semEvA)
    fi_pass(bufsA, fivA)
    issue_gather(fivA, gbufA, semGA)
    start_ev(1, bufsB, semEvB)

    def _pair(ci2, _):
        c0i = 2 * ci2
        c1i = 2 * ci2 + 1
        wait_gather(fivA, gbufA, semGA)
        wait_ev(c1i, bufsB, semEvB)
        fi_pass(bufsB, fivB)
        issue_gather(fivB, gbufB, semGB)

        @pl.when(ci2 > 0)
        def _():
            wait_scatter(sbufsA, semScA)
        main_pass(bufsA, gbufA, sbufsA)
        issue_scatter(sbufsA, semScA)

        @pl.when(ci2 < NPAIR - 1)
        def _():
            start_ev(c0i + 2, bufsA, semEvA)

        wait_gather(fivB, gbufB, semGB)

        @pl.when(ci2 < NPAIR - 1)
        def _():
            wait_ev(c0i + 2, bufsA, semEvA)
            fi_pass(bufsA, fivA)
            issue_gather(fivA, gbufA, semGA)

        @pl.when(ci2 > 0)
        def _():
            wait_scatter(sbufsB, semScB)
        main_pass(bufsB, gbufB, sbufsB)
        issue_scatter(sbufsB, semScB)

        @pl.when(ci2 < NPAIR - 1)
        def _():
            start_ev(c1i + 2, bufsB, semEvB)
        return 0
    lax.fori_loop(0, NPAIR, _pair, 0)

    wait_scatter(sbufsA, semScA)
    wait_scatter(sbufsB, semScB)

    plsc.subcore_barrier()

    for k in range(2):
        asl = pl.ds(sid * SLICE + k * RHALF, RHALF)
        stw = vwA.at[pl.ds(0, RHALF)]
        stt = vwtA.at[pl.ds(0, RHALF)]
        pltpu.sync_copy(accw_sh.at[asl], stw)
        pltpu.sync_copy(stw, wsum_hbm.at[cid, asl])
        pltpu.sync_copy(acct_sh.at[asl], stt)
        pltpu.sync_copy(stt, tsum_hbm.at[cid, asl])


_sc_kernel = functools.partial(
    pl.kernel,
    out_type=(jax.ShapeDtypeStruct((NC, VOLP), jnp.float32),
              jax.ShapeDtypeStruct((NC, VOLP), jnp.float32)),
    mesh=plsc.VectorSubcoreMesh(core_axis_name="c", subcore_axis_name="s"),
    scratch_types=[
        pltpu.VMEM((C,), jnp.float32),
        pltpu.VMEM((C,), jnp.float32),
        pltpu.VMEM((C,), jnp.float32),
        pltpu.VMEM((C,), jnp.float32),
        pltpu.VMEM((C,), jnp.float32),
        pltpu.VMEM((C,), jnp.float32),
        pltpu.VMEM((C,), jnp.float32),
        pltpu.VMEM((C,), jnp.float32),
        pltpu.VMEM((C,), jnp.int32),
        pltpu.VMEM((C,), jnp.int32),
        pltpu.VMEM((C,), jnp.int32),
        pltpu.VMEM((C,), jnp.int32),
        pltpu.VMEM((4 * C,), jnp.int32),
        pltpu.VMEM((4 * C,), jnp.float32),
        pltpu.VMEM((4 * C,), jnp.float32),
        pltpu.VMEM((4 * C,), jnp.int32),
        pltpu.VMEM((4 * C,), jnp.float32),
        pltpu.VMEM((4 * C,), jnp.float32),
        pltpu.VMEM((2, 16), jnp.float32),
        pltpu.SemaphoreType.DMA,
        pltpu.SemaphoreType.DMA,
        pltpu.SemaphoreType.DMA,
        pltpu.SemaphoreType.DMA,
        pltpu.SemaphoreType.DMA,
        pltpu.SemaphoreType.DMA,
        pltpu.VMEM_SHARED((FPAD,), jnp.int32),
        pltpu.VMEM_SHARED((VOLP,), jnp.float32),
        pltpu.VMEM_SHARED((VOLP,), jnp.float32),
    ],
)(_sc_body)


def _combine_body(w_ref, t_ref, o_ref):
    w2 = w_ref[...]
    t2 = t_ref[...]
    o_ref[0, 0] = jnp.sum((t2[0] + t2[1]) / (w2[0] + w2[1] + 1e-8))


_combine = pl.pallas_call(
    _combine_body,
    out_shape=jax.ShapeDtypeStruct((1, 1), jnp.float32),
    out_specs=pl.BlockSpec(memory_space=pltpu.SMEM),
)


def kernel(events, end_flow):
    w = events[:, 0]
    h = events[:, 1]
    t = events[:, 2]
    p = events[:, 3]
    t0 = events[0, 2]
    tl = events[N - 1, 2] - t0
    scal = jnp.stack([jnp.full((16,), t0, jnp.float32),
                      jnp.full((16,), 1.0 / tl, jnp.float32)])
    fp = end_flow.reshape(2, HW)
    fb = fp[:, :FCOL].astype(jnp.bfloat16)
    fu = jax.lax.bitcast_convert_type(fb, jnp.uint16).astype(jnp.uint32)
    packed = jax.lax.bitcast_convert_type(fu[0] | (fu[1] << 16), jnp.int32)
    flow_pad = jnp.concatenate([packed,
                                jnp.zeros((FPAD - FCOL,), jnp.int32)])
    zeros = jnp.zeros((RHALF,), jnp.float32)
    wsum, tsum = _sc_kernel(w, h, t, p, flow_pad, scal, zeros)
    out = _combine(wsum.reshape(NC, VOLP // 128, 128),
                   tsum.reshape(NC, VOLP // 128, 128))
    return out[0, 0]
